# baseline (speedup 1.0000x reference)
import functools

import jax
import jax.numpy as jnp
from jax import lax
from jax.experimental import pallas as pl
from jax.experimental.pallas import tpu as pltpu
from jax.experimental.pallas import tpu_sc as plsc

_SEQ_LEN = 200
_EMBED_DIM = 64
_NUM_WORKERS = 5
_ROWS_PER_WORKER = _SEQ_LEN // _NUM_WORKERS
_SUPERSET_ROWS = _ROWS_PER_WORKER + 8


def _make_sc_copy():
    mesh = plsc.VectorSubcoreMesh(
        core_axis_name="c", subcore_axis_name="s", num_cores=1
    )

    @functools.partial(
        pl.kernel,
        mesh=mesh,
        out_type=jax.ShapeDtypeStruct((_SEQ_LEN, _EMBED_DIM), jnp.float32),
        scratch_types=[pltpu.VMEM((_SEQ_LEN + 1, _EMBED_DIM), jnp.float32)],
    )
    def sc_copy(table_hbm, out_hbm, buf_vmem):
        sid = lax.axis_index("s")
        base = sid * _ROWS_PER_WORKER

        @pl.when(sid < _NUM_WORKERS - 1)
        def _():
            pltpu.sync_copy(
                table_hbm.at[pl.ds(base, _SUPERSET_ROWS), :],
                buf_vmem.at[pl.ds(0, _SUPERSET_ROWS), :],
            )
            pltpu.sync_copy(
                buf_vmem.at[pl.ds(1, _ROWS_PER_WORKER), :],
                out_hbm.at[pl.ds(base, _ROWS_PER_WORKER), :],
            )

        @pl.when(sid == _NUM_WORKERS - 1)
        def _():
            pltpu.sync_copy(table_hbm, buf_vmem)
            pltpu.sync_copy(
                buf_vmem.at[pl.ds(base + 1, _ROWS_PER_WORKER), :],
                out_hbm.at[pl.ds(base, _ROWS_PER_WORKER), :],
            )

    return sc_copy


_sc_copy = _make_sc_copy()


def kernel(x_item_embeddings, pos_table):
    del x_item_embeddings
    return _sc_copy(pos_table)


# speedup vs baseline: 1.0484x; 1.0484x over previous
import functools

import jax
import jax.numpy as jnp
from jax import lax
from jax.experimental import pallas as pl
from jax.experimental.pallas import tpu as pltpu
from jax.experimental.pallas import tpu_sc as plsc

_SEQ_LEN = 200
_EMBED_DIM = 64


def _make_sc_copy():
    num_workers = 8
    total = _SEQ_LEN * _EMBED_DIM
    per_worker = total // num_workers
    assert per_worker * num_workers == total and per_worker % 8 == 0

    mesh = plsc.VectorSubcoreMesh(
        core_axis_name="c",
        subcore_axis_name="s",
        num_cores=1,
        num_subcores=num_workers,
    )

    @functools.partial(
        pl.kernel,
        mesh=mesh,
        out_type=jax.ShapeDtypeStruct((total,), jnp.float32),
        scratch_types=[pltpu.VMEM((per_worker,), jnp.float32)],
    )
    def sc_copy(table_hbm, out_hbm, buf_vmem):
        base = lax.axis_index("s") * per_worker
        pltpu.sync_copy(table_hbm.at[pl.ds(_EMBED_DIM + base, per_worker)], buf_vmem)
        pltpu.sync_copy(buf_vmem, out_hbm.at[pl.ds(base, per_worker)])

    return sc_copy


_sc_copy = _make_sc_copy()


def kernel(x_item_embeddings, pos_table):
    del x_item_embeddings
    flat = pos_table.reshape(-1)
    out = _sc_copy(flat)
    return out.reshape(_SEQ_LEN, _EMBED_DIM)
